# baseline (speedup 1.0000x reference)
import math

import jax
import jax.numpy as jnp
from jax.experimental import pallas as pl

TOP_K = 8


def _router_body(x_ref, w_ref, n_ref, tkw_ref, tki_ref, g_ref, l_ref, lp_ref):
    bt, num_experts = l_ref.shape
    sigma = 1.0 / num_experts

    x = x_ref[...]
    w = w_ref[...]
    logits = jax.lax.dot_general(
        x, w, (((1,), (1,)), ((), ())), preferred_element_type=jnp.float32
    )
    l_ref[...] = logits

    m = jnp.max(logits, axis=1, keepdims=True)
    e = jnp.exp(logits - m)
    s = jnp.sum(e, axis=1, keepdims=True)
    g = e / s
    g_ref[...] = g

    iota = jax.lax.broadcasted_iota(jnp.int32, (bt, num_experts), 1)

    vals = g
    ws = []
    idxs = []
    for _ in range(TOP_K):
        mj = jnp.max(vals, axis=1, keepdims=True)
        ij = jnp.min(
            jnp.where(vals == mj, iota, num_experts), axis=1, keepdims=True
        )
        ws.append(mj)
        idxs.append(ij)
        vals = jnp.where(iota == ij, -jnp.inf, vals)
    w8 = jnp.concatenate(ws, axis=1)
    i8 = jnp.concatenate(idxs, axis=1)
    w8 = w8 / (jnp.sum(w8, axis=1, keepdims=True) + 1e-09)
    tkw_ref[...] = w8
    tki_ref[...] = i8

    nv = logits + n_ref[...] * sigma
    for _ in range(TOP_K - 1):
        mj = jnp.max(nv, axis=1, keepdims=True)
        ij = jnp.min(
            jnp.where(nv == mj, iota, num_experts), axis=1, keepdims=True
        )
        nv = jnp.where(iota == ij, -jnp.inf, nv)
    tau = jnp.max(nv, axis=1, keepdims=True)

    z = (tau - logits) / sigma
    lp_ref[...] = 1.0 - 0.5 * (1.0 + jax.lax.erf(z * (1.0 / math.sqrt(2.0))))


def kernel(x, W, noise):
    n_tokens, hidden = x.shape
    num_experts = W.shape[0]
    bt = 256
    grid = (n_tokens // bt,)

    out_shapes = (
        jax.ShapeDtypeStruct((n_tokens, TOP_K), jnp.float32),
        jax.ShapeDtypeStruct((n_tokens, TOP_K), jnp.int32),
        jax.ShapeDtypeStruct((n_tokens, num_experts), jnp.float32),
        jax.ShapeDtypeStruct((n_tokens, num_experts), jnp.float32),
        jax.ShapeDtypeStruct((n_tokens, num_experts), jnp.float32),
    )
    row_spec = lambda d: pl.BlockSpec((bt, d), lambda i: (i, 0))
    tkw, tki, g, logits, lp = pl.pallas_call(
        _router_body,
        grid=grid,
        in_specs=[
            row_spec(hidden),
            pl.BlockSpec((num_experts, hidden), lambda i: (0, 0)),
            row_spec(num_experts),
        ],
        out_specs=(
            row_spec(TOP_K),
            row_spec(TOP_K),
            row_spec(num_experts),
            row_spec(num_experts),
            row_spec(num_experts),
        ),
        out_shape=out_shapes,
    )(x, W, noise)
    return (tkw, tki, g, logits, lp, tki)


# speedup vs baseline: 1.3515x; 1.3515x over previous
import math

import jax
import jax.numpy as jnp
from jax.experimental import pallas as pl

TOP_K = 8


def _pack_keys(v, iota, rev_iota):
    b = jax.lax.bitcast_convert_type(v, jnp.int32)
    low = jnp.where(b < 0, iota, rev_iota)
    return jax.lax.bitcast_convert_type((b & jnp.int32(~63)) | low, jnp.float32)


def _unpack(keys):
    b = jax.lax.bitcast_convert_type(keys, jnp.int32)
    low = b & jnp.int32(63)
    idx = jnp.where(b < 0, low, jnp.int32(63) - low)
    vals = jax.lax.bitcast_convert_type(b & jnp.int32(~63), jnp.float32)
    return vals, idx


def _router_body(x_ref, w_ref, n_ref, tkw_ref, tki_ref, g_ref, l_ref, lp_ref):
    bt, num_experts = l_ref.shape
    sigma = 1.0 / num_experts

    logits = jax.lax.dot_general(
        x_ref[...], w_ref[...], (((1,), (1,)), ((), ())),
        preferred_element_type=jnp.float32,
    )
    l_ref[...] = logits

    m = jnp.max(logits, axis=1, keepdims=True)
    e = jnp.exp(logits - m)
    s = jnp.sum(e, axis=1, keepdims=True)
    g_ref[...] = e / s

    iota = jax.lax.broadcasted_iota(jnp.int32, (bt, num_experts), 1)
    rev_iota = jnp.int32(num_experts - 1) - iota

    keys = _pack_keys(logits, iota, rev_iota)
    maxes = []
    for _ in range(TOP_K):
        mj = jnp.max(keys, axis=1, keepdims=True)
        maxes.append(mj)
        keys = jnp.where(keys == mj, -jnp.inf, keys)
    k8 = jnp.concatenate(maxes, axis=1)
    v8, i8 = _unpack(k8)
    tki_ref[...] = i8
    e8 = jnp.exp(v8 - m)
    tkw_ref[...] = e8 / (jnp.sum(e8, axis=1, keepdims=True) + s * 1e-09)

    nkeys = _pack_keys(logits + n_ref[...] * sigma, iota, rev_iota)
    for _ in range(TOP_K - 1):
        mj = jnp.max(nkeys, axis=1, keepdims=True)
        nkeys = jnp.where(nkeys == mj, -jnp.inf, nkeys)
    tau, _ = _unpack(jnp.max(nkeys, axis=1, keepdims=True))

    z = (tau - logits) * num_experts
    lp_ref[...] = 0.5 * (1.0 - jax.lax.erf(z * (1.0 / math.sqrt(2.0))))


def kernel(x, W, noise):
    n_tokens, hidden = x.shape
    num_experts = W.shape[0]
    bt = 256
    grid = (n_tokens // bt,)

    out_shapes = (
        jax.ShapeDtypeStruct((n_tokens, TOP_K), jnp.float32),
        jax.ShapeDtypeStruct((n_tokens, TOP_K), jnp.int32),
        jax.ShapeDtypeStruct((n_tokens, num_experts), jnp.float32),
        jax.ShapeDtypeStruct((n_tokens, num_experts), jnp.float32),
        jax.ShapeDtypeStruct((n_tokens, num_experts), jnp.float32),
    )
    row_spec = lambda d: pl.BlockSpec((bt, d), lambda i: (i, 0))
    tkw, tki, g, logits, lp = pl.pallas_call(
        _router_body,
        grid=grid,
        in_specs=[
            row_spec(hidden),
            pl.BlockSpec((num_experts, hidden), lambda i: (0, 0)),
            row_spec(num_experts),
        ],
        out_specs=(
            row_spec(TOP_K),
            row_spec(TOP_K),
            row_spec(num_experts),
            row_spec(num_experts),
            row_spec(num_experts),
        ),
        out_shape=out_shapes,
    )(x, W, noise)
    return (tkw, tki, g, logits, lp, tki)


# speedup vs baseline: 1.5783x; 1.1678x over previous
import math

import jax
import jax.numpy as jnp
from jax.experimental import pallas as pl

TOP_K = 8


def _pack_keys(v, iota, rev_iota):
    b = jax.lax.bitcast_convert_type(v, jnp.int32)
    low = jnp.where(b < 0, iota, rev_iota)
    return jax.lax.bitcast_convert_type((b & jnp.int32(~63)) | low, jnp.float32)


def _unpack(keys):
    b = jax.lax.bitcast_convert_type(keys, jnp.int32)
    low = b & jnp.int32(63)
    idx = jnp.where(b < 0, low, jnp.int32(63) - low)
    vals = jax.lax.bitcast_convert_type(b & jnp.int32(~63), jnp.float32)
    return vals, idx


def _router_body(x_ref, w_ref, n_ref, tkw_ref, tki_ref, g_ref, l_ref, lp_ref):
    bt, num_experts = l_ref.shape
    sigma = 1.0 / num_experts

    logits = jax.lax.dot_general(
        x_ref[...], w_ref[...], (((1,), (1,)), ((), ())),
        preferred_element_type=jnp.float32,
    )
    l_ref[...] = logits

    m = jnp.max(logits, axis=1, keepdims=True)
    e = jnp.exp(logits - m)
    s = jnp.sum(e, axis=1, keepdims=True)
    g_ref[...] = e / s

    iota = jax.lax.broadcasted_iota(jnp.int32, (bt, num_experts), 1)
    rev_iota = jnp.int32(num_experts - 1) - iota

    keys = _pack_keys(logits, iota, rev_iota)
    maxes = []
    for _ in range(TOP_K):
        mj = jnp.max(keys, axis=1, keepdims=True)
        maxes.append(mj)
        keys = jnp.where(keys == mj, -jnp.inf, keys)
    k8 = jnp.concatenate(maxes, axis=1)
    v8, i8 = _unpack(k8)
    tki_ref[...] = i8
    e8 = jnp.exp(v8 - m)
    tkw_ref[...] = e8 / (jnp.sum(e8, axis=1, keepdims=True) + s * 1e-09)

    nkeys = _pack_keys(logits + n_ref[...] * sigma, iota, rev_iota)
    for _ in range(TOP_K - 1):
        mj = jnp.max(nkeys, axis=1, keepdims=True)
        nkeys = jnp.where(nkeys == mj, -jnp.inf, nkeys)
    tau, _ = _unpack(jnp.max(nkeys, axis=1, keepdims=True))

    z = (tau - logits) * num_experts
    lp_ref[...] = 0.5 * (1.0 - jax.lax.erf(z * (1.0 / math.sqrt(2.0))))


def kernel(x, W, noise):
    n_tokens, hidden = x.shape
    num_experts = W.shape[0]
    bt = 512
    grid = (n_tokens // bt,)

    out_shapes = (
        jax.ShapeDtypeStruct((n_tokens, TOP_K), jnp.float32),
        jax.ShapeDtypeStruct((n_tokens, TOP_K), jnp.int32),
        jax.ShapeDtypeStruct((n_tokens, num_experts), jnp.float32),
        jax.ShapeDtypeStruct((n_tokens, num_experts), jnp.float32),
        jax.ShapeDtypeStruct((n_tokens, num_experts), jnp.float32),
    )
    row_spec = lambda d: pl.BlockSpec((bt, d), lambda i: (i, 0))
    tkw, tki, g, logits, lp = pl.pallas_call(
        _router_body,
        grid=grid,
        in_specs=[
            row_spec(hidden),
            pl.BlockSpec((num_experts, hidden), lambda i: (0, 0)),
            row_spec(num_experts),
        ],
        out_specs=(
            row_spec(TOP_K),
            row_spec(TOP_K),
            row_spec(num_experts),
            row_spec(num_experts),
            row_spec(num_experts),
        ),
        out_shape=out_shapes,
    )(x, W, noise)
    return (tkw, tki, g, logits, lp, tki)


# speedup vs baseline: 1.6395x; 1.0388x over previous
import math

import jax
import jax.numpy as jnp
from jax.experimental import pallas as pl

TOP_K = 8


def _pack_keys(v, iota, rev_iota):
    b = jax.lax.bitcast_convert_type(v, jnp.int32)
    low = jnp.where(b < 0, iota, rev_iota)
    return jax.lax.bitcast_convert_type((b & jnp.int32(~63)) | low, jnp.float32)


def _unpack(keys):
    b = jax.lax.bitcast_convert_type(keys, jnp.int32)
    low = b & jnp.int32(63)
    idx = jnp.where(b < 0, low, jnp.int32(63) - low)
    vals = jax.lax.bitcast_convert_type(b & jnp.int32(~63), jnp.float32)
    return vals, idx


def _router_body(x_ref, w_ref, n_ref, tkw_ref, tki_ref, g_ref, l_ref, lp_ref):
    bt, num_experts = l_ref.shape
    sigma = 1.0 / num_experts

    logits = jax.lax.dot_general(
        x_ref[...], w_ref[...], (((1,), (1,)), ((), ())),
        preferred_element_type=jnp.float32,
    )
    l_ref[...] = logits

    m = jnp.max(logits, axis=1, keepdims=True)
    e = jnp.exp(logits - m)
    s = jnp.sum(e, axis=1, keepdims=True)
    g_ref[...] = e / s

    iota = jax.lax.broadcasted_iota(jnp.int32, (bt, num_experts), 1)
    rev_iota = jnp.int32(num_experts - 1) - iota

    keys = _pack_keys(logits, iota, rev_iota)
    maxes = []
    for _ in range(TOP_K):
        mj = jnp.max(keys, axis=1, keepdims=True)
        maxes.append(mj)
        keys = jnp.where(keys == mj, -jnp.inf, keys)
    k8 = jnp.concatenate(maxes, axis=1)
    v8, i8 = _unpack(k8)
    tki_ref[...] = i8
    e8 = jnp.exp(v8 - m)
    tkw_ref[...] = e8 / (jnp.sum(e8, axis=1, keepdims=True) + s * 1e-09)

    nkeys = _pack_keys(logits + n_ref[...] * sigma, iota, rev_iota)
    for _ in range(TOP_K - 1):
        mj = jnp.max(nkeys, axis=1, keepdims=True)
        nkeys = jnp.where(nkeys == mj, -jnp.inf, nkeys)
    tau, _ = _unpack(jnp.max(nkeys, axis=1, keepdims=True))

    z = (tau - logits) * num_experts
    lp_ref[...] = 0.5 * (1.0 - jax.lax.erf(z * (1.0 / math.sqrt(2.0))))


def kernel(x, W, noise):
    n_tokens, hidden = x.shape
    num_experts = W.shape[0]
    bt = 1024
    grid = (n_tokens // bt,)

    out_shapes = (
        jax.ShapeDtypeStruct((n_tokens, TOP_K), jnp.float32),
        jax.ShapeDtypeStruct((n_tokens, TOP_K), jnp.int32),
        jax.ShapeDtypeStruct((n_tokens, num_experts), jnp.float32),
        jax.ShapeDtypeStruct((n_tokens, num_experts), jnp.float32),
        jax.ShapeDtypeStruct((n_tokens, num_experts), jnp.float32),
    )
    row_spec = lambda d: pl.BlockSpec((bt, d), lambda i: (i, 0))
    tkw, tki, g, logits, lp = pl.pallas_call(
        _router_body,
        grid=grid,
        in_specs=[
            row_spec(hidden),
            pl.BlockSpec((num_experts, hidden), lambda i: (0, 0)),
            row_spec(num_experts),
        ],
        out_specs=(
            row_spec(TOP_K),
            row_spec(TOP_K),
            row_spec(num_experts),
            row_spec(num_experts),
            row_spec(num_experts),
        ),
        out_shape=out_shapes,
    )(x, W, noise)
    return (tkw, tki, g, logits, lp, tki)
